# baseline (speedup 1.0000x reference)
import jax
import jax.numpy as jnp
from jax import lax
from jax.experimental import pallas as pl
from jax.experimental.pallas import tpu as pltpu

_L = 16384
_NCORE = 2
_NCHUNK = (512 * 512) // _L // _NCORE
_ROWS_PER_CHUNK = _L // 512
_ROWS_PER_CORE = 512 // _NCORE

_DN = (((1,), (1,)), ((), ()))


def _body(x_ref, w1_ref, w2_ref, w3_ref, b1_ref, b2_ref, b3_ref,
          s3_ref, cnt_ref, h1s_ref, h2s_ref):
    c = pl.program_id(0)
    w_i = w1_ref[:, 0:1]
    w_j = w1_ref[:, 1:2]
    w_v = w1_ref[:, 2:3]
    w2a = jnp.concatenate(
        [w2_ref[...], b2_ref[...].reshape(32, 1),
         jnp.zeros((32, 7), jnp.float32)], axis=1).astype(jnp.bfloat16)
    w3a = jnp.concatenate(
        [w3_ref[...], b3_ref[...].reshape(64, 1),
         jnp.zeros((64, 7), jnp.float32)], axis=1).astype(jnp.bfloat16)
    h1s_ref[16:24, :] = jnp.zeros((8, _L), jnp.bfloat16)
    h1s_ref[16:17, :] = jnp.ones((1, _L), jnp.bfloat16)
    h2s_ref[32:40, :] = jnp.zeros((8, _L), jnp.bfloat16)
    h2s_ref[32:33, :] = jnp.ones((1, _L), jnp.bfloat16)

    t = lax.broadcasted_iota(jnp.int32, (1, _L), 1)
    fj = (t & 511).astype(jnp.float32)
    fi0 = (t >> 9).astype(jnp.float32)
    q = w_i * fi0 + w_j * fj + b1_ref[...].reshape(16, 1)
    row0 = (c * _ROWS_PER_CORE).astype(jnp.float32)

    def chunk(k, carry):
        s3, cnt = carry
        v = x_ref[pl.ds(k * _ROWS_PER_CHUNK, _ROWS_PER_CHUNK), :].reshape(
            1, _L)
        dq = w_i * (row0 + jnp.float32(_ROWS_PER_CHUNK * k))
        h1 = jnp.maximum(w_v * v + q + dq, 0.0).astype(jnp.bfloat16)
        h1s_ref[0:16, :] = h1
        h2 = jnp.maximum(
            jnp.dot(w2a, h1s_ref[...],
                    preferred_element_type=jnp.float32),
            0.0).astype(jnp.bfloat16)
        h2s_ref[0:32, :] = h2
        h3 = jnp.maximum(
            jnp.dot(w3a, h2s_ref[...],
                    preferred_element_type=jnp.float32), 0.0)
        mf = (v != 0.0).astype(jnp.float32)
        s3 = s3 + lax.dot_general(h3, mf, _DN,
                                  preferred_element_type=jnp.float32)
        cnt = cnt + jnp.sum(mf)
        return s3, cnt

    carry = (jnp.zeros((64, 1), jnp.float32), jnp.float32(0.0))
    for k in range(_NCHUNK):
        carry = chunk(k, carry)
    s3, cnt = carry
    s3_ref[...] = s3.reshape(1, 64, 1)
    cnt_ref[...] = cnt.reshape(1, 1, 1)


def kernel(x, W1, b1, W2, b2, W3, b3, W4, b4):
    s3p, cntp = pl.pallas_call(
        _body,
        grid=(_NCORE,),
        in_specs=[
            pl.BlockSpec((_ROWS_PER_CORE, 512), lambda c: (c, 0)),
            pl.BlockSpec((16, 3), lambda c: (0, 0)),
            pl.BlockSpec((32, 16), lambda c: (0, 0)),
            pl.BlockSpec((64, 32), lambda c: (0, 0)),
            pl.BlockSpec((16,), lambda c: (0,)),
            pl.BlockSpec((32,), lambda c: (0,)),
            pl.BlockSpec((64,), lambda c: (0,)),
        ],
        out_specs=[
            pl.BlockSpec((1, 64, 1), lambda c: (c, 0, 0)),
            pl.BlockSpec((1, 1, 1), lambda c: (c, 0, 0)),
        ],
        out_shape=[
            jax.ShapeDtypeStruct((_NCORE, 64, 1), jnp.float32),
            jax.ShapeDtypeStruct((_NCORE, 1, 1), jnp.float32),
        ],
        scratch_shapes=[pltpu.VMEM((24, _L), jnp.bfloat16),
                        pltpu.VMEM((40, _L), jnp.bfloat16)],
        compiler_params=pltpu.CompilerParams(
            dimension_semantics=("parallel",)),
    )(x, W1, W2, W3, b1, b2, b3)
    s3 = jnp.sum(s3p, axis=0)
    cnt = jnp.sum(cntp)
    out = lax.dot_general(s3 / cnt, W4, (((0,), (1,)), ((), ())),
                          preferred_element_type=jnp.float32)
    return out + b4.reshape(1, 128)


# speedup vs baseline: 1.1071x; 1.1071x over previous
import jax
import jax.numpy as jnp
from jax import lax
from jax.experimental import pallas as pl
from jax.experimental.pallas import tpu as pltpu

_L = 16384
_NCHUNK = (512 * 512) // _L
_ROWS_PER_CHUNK = _L // 512

_DN = (((1,), (1,)), ((), ()))


def _body(x_ref, w1_ref, w2_ref, w3_ref, w4_ref, b1_ref, b2_ref, b3_ref,
          b4_ref, out_ref, xv_ref, h1s_ref, h2s_ref, sem):
    cp0 = pltpu.make_async_copy(x_ref.at[0:256, :], xv_ref.at[0:256, :],
                                sem.at[0])
    cp1 = pltpu.make_async_copy(x_ref.at[256:512, :], xv_ref.at[256:512, :],
                                sem.at[1])
    cp0.start()
    cp1.start()
    w_i = w1_ref[:, 0:1]
    w_j = w1_ref[:, 1:2]
    w_v = w1_ref[:, 2:3]
    w2a = jnp.concatenate(
        [w2_ref[...], b2_ref[...].reshape(32, 1),
         jnp.zeros((32, 7), jnp.float32)], axis=1).astype(jnp.bfloat16)
    w3a = jnp.concatenate(
        [w3_ref[...], b3_ref[...].reshape(64, 1),
         jnp.zeros((64, 7), jnp.float32)], axis=1).astype(jnp.bfloat16)
    h1s_ref[16:24, :] = jnp.zeros((8, _L), jnp.bfloat16)
    h1s_ref[16:17, :] = jnp.ones((1, _L), jnp.bfloat16)
    h2s_ref[32:40, :] = jnp.zeros((8, _L), jnp.bfloat16)
    h2s_ref[32:33, :] = jnp.ones((1, _L), jnp.bfloat16)

    t = lax.broadcasted_iota(jnp.int32, (1, _L), 1)
    fj = (t & 511).astype(jnp.float32)
    fi0 = (t >> 9).astype(jnp.float32)
    q = w_i * fi0 + w_j * fj + b1_ref[...].reshape(16, 1)

    def chunk(k, carry):
        s3, cnt = carry
        v = xv_ref[pl.ds(k * _ROWS_PER_CHUNK, _ROWS_PER_CHUNK), :].reshape(
            1, _L)
        dq = w_i * jnp.float32(_ROWS_PER_CHUNK * k)
        h1 = jnp.maximum(w_v * v + q + dq, 0.0).astype(jnp.bfloat16)
        h1s_ref[0:16, :] = h1
        h2 = jnp.maximum(
            jnp.dot(w2a, h1s_ref[...],
                    preferred_element_type=jnp.float32),
            0.0).astype(jnp.bfloat16)
        h2s_ref[0:32, :] = h2
        h3 = jnp.maximum(
            jnp.dot(w3a, h2s_ref[...],
                    preferred_element_type=jnp.float32), 0.0)
        mf = (v != 0.0).astype(jnp.float32)
        s3 = s3 + lax.dot_general(h3, mf, _DN,
                                  preferred_element_type=jnp.float32)
        cnt = cnt + jnp.sum(mf)
        return s3, cnt

    carry = (jnp.zeros((64, 1), jnp.float32), jnp.float32(0.0))
    cp0.wait()
    for k in range(_NCHUNK // 2):
        carry = chunk(k, carry)
    cp1.wait()
    for k in range(_NCHUNK // 2, _NCHUNK):
        carry = chunk(k, carry)
    s3, cnt = carry

    out = lax.dot_general(s3 / cnt, w4_ref[...], (((0,), (1,)), ((), ())),
                          preferred_element_type=jnp.float32)
    out_ref[...] = out + b4_ref[...].reshape(1, 128)


def kernel(x, W1, b1, W2, b2, W3, b3, W4, b4):
    n = 9
    return pl.pallas_call(
        _body,
        out_shape=jax.ShapeDtypeStruct((1, 128), jnp.float32),
        in_specs=[pl.BlockSpec(memory_space=pltpu.MemorySpace.HBM)]
        + [pl.BlockSpec(memory_space=pltpu.MemorySpace.VMEM)] * (n - 1),
        scratch_shapes=[pltpu.VMEM((512, 512), jnp.float32),
                        pltpu.VMEM((24, _L), jnp.bfloat16),
                        pltpu.VMEM((40, _L), jnp.bfloat16),
                        pltpu.SemaphoreType.DMA((2,))],
    )(x, W1, W2, W3, W4, b1, b2, b3, b4)


# speedup vs baseline: 1.1499x; 1.0387x over previous
import jax
import jax.numpy as jnp
from jax import lax
from jax.experimental import pallas as pl
from jax.experimental.pallas import tpu as pltpu

_L = 16384
_NCHUNK = (512 * 512) // _L
_ROWS_PER_CHUNK = _L // 512

_DN = (((1,), (1,)), ((), ()))


def _body(x_ref, w1_ref, w2_ref, w3_ref, w4_ref, b1_ref, b2_ref, b3_ref,
          b4_ref, out_ref, h1s_ref, h2s_ref):
    w_i = w1_ref[:, 0:1]
    w_j = w1_ref[:, 1:2]
    w_v = w1_ref[:, 2:3]
    w2a = jnp.concatenate(
        [w2_ref[...], b2_ref[...].reshape(32, 1),
         jnp.zeros((32, 7), jnp.float32)], axis=1).astype(jnp.bfloat16)
    w3a = jnp.concatenate(
        [w3_ref[...], b3_ref[...].reshape(64, 1),
         jnp.zeros((64, 7), jnp.float32)], axis=1).astype(jnp.bfloat16)
    h1s_ref[16:24, :] = jnp.zeros((8, _L), jnp.bfloat16)
    h1s_ref[16:17, :] = jnp.ones((1, _L), jnp.bfloat16)
    h2s_ref[32:40, :] = jnp.zeros((8, _L), jnp.bfloat16)
    h2s_ref[32:33, :] = jnp.ones((1, _L), jnp.bfloat16)

    t = lax.broadcasted_iota(jnp.int32, (1, _L), 1)
    fj = (t & 511).astype(jnp.float32)
    fi0 = (t >> 9).astype(jnp.float32)
    q = w_i * fi0 + w_j * fj + b1_ref[...].reshape(16, 1)

    def chunk(k, carry):
        s3, cnt = carry
        v = x_ref[pl.ds(k * _ROWS_PER_CHUNK, _ROWS_PER_CHUNK), :].reshape(
            1, _L)
        dq = w_i * jnp.float32(_ROWS_PER_CHUNK * k)
        h1 = jnp.maximum(w_v * v + q + dq, 0.0).astype(jnp.bfloat16)
        h1s_ref[0:16, :] = h1
        h2 = jnp.maximum(
            jnp.dot(w2a, h1s_ref[...],
                    preferred_element_type=jnp.float32),
            0.0).astype(jnp.bfloat16)
        h2s_ref[0:32, :] = h2
        h3 = jnp.maximum(
            jnp.dot(w3a, h2s_ref[...],
                    preferred_element_type=jnp.float32), 0.0)
        mf = (v != 0.0).astype(jnp.float32)
        s3 = s3 + lax.dot_general(h3, mf, _DN,
                                  preferred_element_type=jnp.float32)
        cnt = cnt + jnp.sum(mf)
        return s3, cnt

    carry = (jnp.zeros((64, 1), jnp.float32), jnp.float32(0.0))
    for k in range(_NCHUNK):
        carry = chunk(k, carry)
    s3, cnt = carry

    out = lax.dot_general(s3 / cnt, w4_ref[...], (((0,), (1,)), ((), ())),
                          preferred_element_type=jnp.float32)
    out_ref[...] = out + b4_ref[...].reshape(1, 128)


def kernel(x, W1, b1, W2, b2, W3, b3, W4, b4):
    return pl.pallas_call(
        _body,
        out_shape=jax.ShapeDtypeStruct((1, 128), jnp.float32),
        scratch_shapes=[pltpu.VMEM((24, _L), jnp.bfloat16),
                        pltpu.VMEM((40, _L), jnp.bfloat16)],
    )(x, W1, W2, W3, W4, b1, b2, b3, b4)


# speedup vs baseline: 1.1748x; 1.0217x over previous
import jax
import jax.numpy as jnp
from jax import lax
from jax.experimental import pallas as pl
from jax.experimental.pallas import tpu as pltpu

_L = 8192
_NCHUNK = (512 * 512) // _L
_ROWS_PER_CHUNK = _L // 512

_DN = (((1,), (1,)), ((), ()))


def _body(x_ref, w1_ref, w2_ref, w3_ref, w4_ref, b1_ref, b2_ref, b3_ref,
          b4_ref, out_ref, h1s_ref, h2s_ref):
    w_i = w1_ref[:, 0:1]
    w_j = w1_ref[:, 1:2]
    w_v = w1_ref[:, 2:3]
    w2a = jnp.concatenate(
        [w2_ref[...], b2_ref[...].reshape(32, 1),
         jnp.zeros((32, 7), jnp.float32)], axis=1).astype(jnp.bfloat16)
    w3a = jnp.concatenate(
        [w3_ref[...], b3_ref[...].reshape(64, 1),
         jnp.zeros((64, 7), jnp.float32)], axis=1).astype(jnp.bfloat16)
    h1s_ref[16:24, :] = jnp.zeros((8, _L), jnp.bfloat16)
    h1s_ref[16:17, :] = jnp.ones((1, _L), jnp.bfloat16)
    h2s_ref[32:40, :] = jnp.zeros((8, _L), jnp.bfloat16)
    h2s_ref[32:33, :] = jnp.ones((1, _L), jnp.bfloat16)

    t = lax.broadcasted_iota(jnp.int32, (1, _L), 1)
    fj = (t & 511).astype(jnp.float32)
    fi0 = (t >> 9).astype(jnp.float32)
    q = w_i * fi0 + w_j * fj + b1_ref[...].reshape(16, 1)

    def chunk(k, carry):
        s3, cnt = carry
        v = x_ref[pl.ds(k * _ROWS_PER_CHUNK, _ROWS_PER_CHUNK), :].reshape(
            1, _L)
        dq = w_i * jnp.float32(_ROWS_PER_CHUNK * k)
        h1 = jnp.maximum(w_v * v + q + dq, 0.0).astype(jnp.bfloat16)
        h1s_ref[0:16, :] = h1
        h2 = jnp.maximum(
            jnp.dot(w2a, h1s_ref[...],
                    preferred_element_type=jnp.float32),
            0.0).astype(jnp.bfloat16)
        h2s_ref[0:32, :] = h2
        h3 = jnp.maximum(
            jnp.dot(w3a, h2s_ref[...],
                    preferred_element_type=jnp.float32), 0.0)
        mf = (v != 0.0).astype(jnp.float32)
        s3 = s3 + lax.dot_general(h3, mf, _DN,
                                  preferred_element_type=jnp.float32)
        cnt = cnt + jnp.sum(mf)
        return s3, cnt

    carry = (jnp.zeros((64, 1), jnp.float32), jnp.float32(0.0))
    for k in range(_NCHUNK):
        carry = chunk(k, carry)
    s3, cnt = carry

    out = lax.dot_general(s3 / cnt, w4_ref[...], (((0,), (1,)), ((), ())),
                          preferred_element_type=jnp.float32)
    out_ref[...] = out + b4_ref[...].reshape(1, 128)


def kernel(x, W1, b1, W2, b2, W3, b3, W4, b4):
    return pl.pallas_call(
        _body,
        out_shape=jax.ShapeDtypeStruct((1, 128), jnp.float32),
        scratch_shapes=[pltpu.VMEM((24, _L), jnp.bfloat16),
                        pltpu.VMEM((40, _L), jnp.bfloat16)],
    )(x, W1, W2, W3, W4, b1, b2, b3, b4)
